# baseline (speedup 1.0000x reference)
import functools

import jax
import jax.numpy as jnp
from jax import lax
from jax.experimental import pallas as pl
from jax.experimental.pallas import tpu as pltpu
from jax.experimental.pallas import tpu_sc as plsc

B = 8
N = 512 * 512
NB = 16384
NBR = 136
NBP = NBR * 128
SHIFT = 14
NW = 32
PER_W = N * B // NW
CH = 4096
NCHUNK = PER_W // CH


def _prep_body(l_ref, t_ref, m_ref, p_ref):
    l = l_ref[0, 0]
    t = t_ref[0, 0]
    tpos = jnp.where(t > 0.5, 1.0, 0.0).astype(jnp.float32)
    e = 1.0 - l * (2.0 * tpos - 1.0)
    m_ref[0, 0, :] = jnp.broadcast_to(jnp.max(e), (128,))
    p_ref[0, 0, :] = jnp.broadcast_to(jnp.sum(tpos), (128,))


def _prepass(logits4, targets4):
    return pl.pallas_call(
        _prep_body,
        grid=(B,),
        in_specs=[
            pl.BlockSpec((1, 1, 512, 512), lambda i: (i, 0, 0, 0)),
            pl.BlockSpec((1, 1, 512, 512), lambda i: (i, 0, 0, 0)),
        ],
        out_specs=[
            pl.BlockSpec((1, 1, 128), lambda i: (i, 0, 0)),
            pl.BlockSpec((1, 1, 128), lambda i: (i, 0, 0)),
        ],
        out_shape=[
            jax.ShapeDtypeStruct((B, 1, 128), jnp.float32),
            jax.ShapeDtypeStruct((B, 1, 128), jnp.float32),
        ],
    )(logits4, targets4)


def _sc_hist_body(l_hbm, t_hbm, top_hbm, out_hbm,
                  lb0, lb1, tb0, tb1, topv, h0, h1, h2, sem0, sem1):
    cid = lax.axis_index("c")
    sid = lax.axis_index("s")
    wid = cid * 16 + sid
    img = cid * 4 + sid // 4
    band0 = (sid % 4) * (NCHUNK * 8)

    zeros16 = jnp.zeros((16,), jnp.float32)

    @plsc.parallel_loop(0, NBP // 16, unroll=8)
    def _zero(j):
        h0[pl.ds(j * 16, 16)] = zeros16
        h1[pl.ds(j * 16, 16)] = zeros16
        h2[pl.ds(j * 16, 16)] = zeros16

    pltpu.sync_copy(top_hbm.at[pl.ds(img * 16, 16)], topv)
    top = topv[...]

    iota = lax.iota(jnp.int32, 16)
    nxt_idx = jnp.minimum(iota + 1, 15)
    not_last = iota < 15
    cnt_run = (iota + 1).astype(jnp.float32)
    neg_cnt_run = -cnt_run
    gdn = lax.GatherDimensionNumbers(
        offset_dims=(), collapsed_slice_dims=(0,), start_index_map=(0,))

    lbufs = (lb0, lb1)
    tbufs = (tb0, tb1)
    sems = (sem0, sem1)

    def _start(ci):
        rows = band0 + ci * 8
        lc = pltpu.async_copy(
            l_hbm.at[img, pl.ds(rows, 8), :], lbufs[ci % 2], sems[ci % 2])
        tc_ = pltpu.async_copy(
            t_hbm.at[img, pl.ds(rows, 8), :], tbufs[ci % 2], sems[ci % 2])
        return lc, tc_

    pend = _start(0)
    for ci in range(NCHUNK):
        lbuf = lbufs[ci % 2]
        tbuf = tbufs[ci % 2]
        nxt_pend = _start(ci + 1) if ci + 1 < NCHUNK else None
        pend[0].wait()
        pend[1].wait()
        pend = nxt_pend

        @plsc.parallel_loop(0, CH // 16, unroll=6)
        def _vec(v):
            r = v >> 5
            c = (v & 31) * 16
            l = lbuf[r, pl.ds(c, 16)]
            t = tbuf[r, pl.ds(c, 16)]
            lt = l * t
            e = (1.0 + l) - (lt + lt)
            es0 = jnp.maximum(e, 0.0)
            ebits = lax.bitcast_convert_type(es0, jnp.int32)
            d = lax.shift_right_arithmetic(top - ebits, SHIFT)
            bkt = jnp.minimum(jnp.maximum((NB - 1) - d, 0), NB - 1)
            gi = t.astype(jnp.int32)
            packed = (ebits & -2) | gi
            bs, sp = plsc.sort_key_val(bkt, packed)
            gs = (sp & 1).astype(jnp.float32)
            es = lax.bitcast_convert_type(sp & -2, jnp.float32)
            nxt = lax.gather(bs, nxt_idx[:, None], gdn, slice_sizes=(1,),
                             mode=lax.GatherScatterMode.PROMISE_IN_BOUNDS)
            brk = bs != nxt
            isend = brk | (~not_last)
            issub = brk & not_last
            ce = plsc.cumsum(es)
            cg = plsc.cumsum(gs)
            plsc.addupdate_scatter(h0, [bs], cnt_run, mask=isend)
            plsc.addupdate_scatter(h0, [nxt], neg_cnt_run, mask=issub)
            plsc.addupdate_scatter(h1, [bs], cg, mask=isend)
            plsc.addupdate_scatter(h1, [nxt], -cg, mask=issub)
            plsc.addupdate_scatter(h2, [bs], ce, mask=isend)
            plsc.addupdate_scatter(h2, [nxt], -ce, mask=issub)

    obase = wid * 3 * NBP
    pltpu.sync_copy(h0, out_hbm.at[pl.ds(obase, NBP)])
    pltpu.sync_copy(h1, out_hbm.at[pl.ds(obase + NBP, NBP)])
    pltpu.sync_copy(h2, out_hbm.at[pl.ds(obase + 2 * NBP, NBP)])


def _sc_hist(l3, t3, topflat):
    mesh = plsc.VectorSubcoreMesh(core_axis_name="c", subcore_axis_name="s")
    k = functools.partial(
        pl.kernel,
        mesh=mesh,
        compiler_params=pltpu.CompilerParams(needs_layout_passes=False),
        out_type=jax.ShapeDtypeStruct((NW * 3 * NBP,), jnp.float32),
        scratch_types=[
            pltpu.VMEM((8, 512), jnp.float32),
            pltpu.VMEM((8, 512), jnp.float32),
            pltpu.VMEM((8, 512), jnp.float32),
            pltpu.VMEM((8, 512), jnp.float32),
            pltpu.VMEM((16,), jnp.int32),
            pltpu.VMEM((NBP,), jnp.float32),
            pltpu.VMEM((NBP,), jnp.float32),
            pltpu.VMEM((NBP,), jnp.float32),
            pltpu.SemaphoreType.DMA,
            pltpu.SemaphoreType.DMA,
        ],
    )(_sc_hist_body)
    return k(l3, t3, topflat)


def _fin_body(h_ref, p_ref, m_ref, o_ref):
    f32 = jnp.float32
    r = lax.broadcasted_iota(jnp.int32, (128, 128), 0)
    c = lax.broadcasted_iota(jnp.int32, (128, 128), 1)
    w_incl = (r >= c).astype(f32)
    w_strict = (r > c).astype(f32)

    total = jnp.zeros((), f32)
    for img in range(B):
        cc = img // 4
        s0 = (img % 4) * 4
        rows = [cc * 16 + s0 + j for j in range(4)]
        cnt = jnp.zeros((128, 128), f32)
        cntp = jnp.zeros((128, 128), f32)
        sm = jnp.zeros((128, 128), f32)
        for w in rows:
            cnt = cnt + h_ref[w, 0, :128, :]
            cntp = cntp + h_ref[w, 1, :128, :]
            sm = sm + h_ref[w, 2, :128, :]

        def suffix(x):
            s_in = jnp.dot(x, w_incl, preferred_element_type=f32)
            rowtot = s_in[:, :1]
            above = jnp.dot(w_strict.T, rowtot, preferred_element_type=f32)
            return s_in + above

        suf = suffix(cnt)
        sufp = suffix(cntp)
        a = suf - cnt
        ap = sufp - cntp
        p = p_ref[img, 0, 0]
        m = m_ref[img, 0, 0]
        tp = cntp
        tn = cnt - cntp
        sumpos = sm * cntp / jnp.maximum(cnt, 1.0)
        sumneg = sm - sumpos
        aa = p + a - ap
        contrib = (sumpos + sumneg * (p - ap - tp) / jnp.maximum(aa + tn, 1.0)
                   ) / jnp.maximum(aa, 1.0)
        loss = jnp.sum(contrib)
        loss = jnp.where(p == 0.0, jnp.maximum(m, 0.0), loss)
        total = total + loss
    o_ref[...] = jnp.broadcast_to(total * (1.0 / B), (1, 1))


def _finalize(h4, pc, mc):
    return pl.pallas_call(
        _fin_body,
        in_specs=[
            pl.BlockSpec((NW, 3, NBR, 128), lambda: (0, 0, 0, 0)),
            pl.BlockSpec((B, 1, 128), lambda: (0, 0, 0)),
            pl.BlockSpec((B, 1, 128), lambda: (0, 0, 0)),
        ],
        out_specs=pl.BlockSpec((1, 1), lambda: (0, 0)),
        out_shape=jax.ShapeDtypeStruct((1, 1), jnp.float32),
    )(h4, pc, mc)


def kernel(logits, targets):
    l4 = logits.reshape(B, 1, 512, 512)
    t4 = targets.reshape(B, 1, 512, 512)
    mc, pc = _prepass(l4, t4)
    top = lax.bitcast_convert_type(mc[:, 0, :16], jnp.int32).reshape(B * 16)
    hists = _sc_hist(logits.reshape(B, 512, 512), targets.reshape(B, 512, 512),
                     top)
    h4 = hists.reshape(NW, 3, NBR, 128)
    out = _finalize(h4, pc, mc)
    return out.reshape(())


# speedup vs baseline: 1.2864x; 1.2864x over previous
import functools

import jax
import jax.numpy as jnp
from jax import lax
from jax.experimental import pallas as pl
from jax.experimental.pallas import tpu as pltpu
from jax.experimental.pallas import tpu_sc as plsc

B = 8
N = 512 * 512
NB = 16384
NBR = 136
NBP = NBR * 128
SHIFT = 14
NW = 32
PER_W = N * B // NW
CH = 4096
NCHUNK = PER_W // CH


def _prep_body(l_ref, t_ref, m_ref, p_ref):
    l = l_ref[0, 0]
    t = t_ref[0, 0]
    tpos = jnp.where(t > 0.5, 1.0, 0.0).astype(jnp.float32)
    e = 1.0 - l * (2.0 * tpos - 1.0)
    m_ref[0, 0, :] = jnp.broadcast_to(jnp.max(e), (128,))
    p_ref[0, 0, :] = jnp.broadcast_to(jnp.sum(tpos), (128,))


def _prepass(logits4, targets4):
    return pl.pallas_call(
        _prep_body,
        grid=(B,),
        in_specs=[
            pl.BlockSpec((1, 1, 512, 512), lambda i: (i, 0, 0, 0)),
            pl.BlockSpec((1, 1, 512, 512), lambda i: (i, 0, 0, 0)),
        ],
        out_specs=[
            pl.BlockSpec((1, 1, 128), lambda i: (i, 0, 0)),
            pl.BlockSpec((1, 1, 128), lambda i: (i, 0, 0)),
        ],
        out_shape=[
            jax.ShapeDtypeStruct((B, 1, 128), jnp.float32),
            jax.ShapeDtypeStruct((B, 1, 128), jnp.float32),
        ],
    )(logits4, targets4)


def _sc_hist_body(l_hbm, t_hbm, top_hbm, out_hbm,
                  lb0, lb1, tb0, tb1, topv, h0, h1, h2, sem0, sem1):
    cid = lax.axis_index("c")
    sid = lax.axis_index("s")
    wid = cid * 16 + sid
    img = cid * 4 + sid // 4
    band0 = (sid % 4) * (NCHUNK * 8)

    zeros16 = jnp.zeros((16,), jnp.float32)

    @plsc.parallel_loop(0, NBP // 16, unroll=8)
    def _zero(j):
        h0[pl.ds(j * 16, 16)] = zeros16
        h1[pl.ds(j * 16, 16)] = zeros16
        h2[pl.ds(j * 16, 16)] = zeros16

    pltpu.sync_copy(top_hbm.at[pl.ds(img * 16, 16)], topv)
    top = topv[...]

    iota = lax.iota(jnp.int32, 16)
    nxt_idx = jnp.minimum(iota + 1, 15)
    not_last = iota < 15
    cnt_run = (iota + 1).astype(jnp.float32)
    neg_cnt_run = -cnt_run
    gdn = lax.GatherDimensionNumbers(
        offset_dims=(), collapsed_slice_dims=(0,), start_index_map=(0,))

    lbufs = (lb0, lb1)
    tbufs = (tb0, tb1)
    sems = (sem0, sem1)

    def _start(ci):
        rows = band0 + ci * 8
        lc = pltpu.async_copy(
            l_hbm.at[img, pl.ds(rows, 8), :], lbufs[ci % 2], sems[ci % 2])
        tc_ = pltpu.async_copy(
            t_hbm.at[img, pl.ds(rows, 8), :], tbufs[ci % 2], sems[ci % 2])
        return lc, tc_

    pend = _start(0)
    for ci in range(NCHUNK):
        lbuf = lbufs[ci % 2]
        tbuf = tbufs[ci % 2]
        nxt_pend = _start(ci + 1) if ci + 1 < NCHUNK else None
        pend[0].wait()
        pend[1].wait()
        pend = nxt_pend

        @plsc.parallel_loop(0, CH // 16, unroll=4)
        def _vec(v):
            r = v >> 5
            c = (v & 31) * 16
            l = lbuf[r, pl.ds(c, 16)]
            t = tbuf[r, pl.ds(c, 16)]
            lt = l * t
            e = (1.0 + l) - (lt + lt)
            es0 = jnp.maximum(e, 0.0)
            ebits = lax.bitcast_convert_type(es0, jnp.int32)
            d = lax.shift_right_arithmetic(top - ebits, SHIFT)
            bkt = jnp.minimum(jnp.maximum((NB - 1) - d, 0), NB - 1)
            bs, es = plsc.sort_key_val(bkt, es0)
            _, gs = plsc.sort_key_val(bkt, t)
            nxt = lax.gather(bs, nxt_idx[:, None], gdn, slice_sizes=(1,),
                             mode=lax.GatherScatterMode.PROMISE_IN_BOUNDS)
            brk = bs != nxt
            isend = brk | (~not_last)
            issub = brk & not_last
            ce = plsc.cumsum(es)
            cg = plsc.cumsum(gs)
            plsc.addupdate_scatter(h0, [bs], cnt_run, mask=isend)
            plsc.addupdate_scatter(h0, [nxt], neg_cnt_run, mask=issub)
            plsc.addupdate_scatter(h1, [bs], cg, mask=isend)
            plsc.addupdate_scatter(h1, [nxt], -cg, mask=issub)
            plsc.addupdate_scatter(h2, [bs], ce, mask=isend)
            plsc.addupdate_scatter(h2, [nxt], -ce, mask=issub)

    obase = wid * 3 * NBP
    pltpu.sync_copy(h0, out_hbm.at[pl.ds(obase, NBP)])
    pltpu.sync_copy(h1, out_hbm.at[pl.ds(obase + NBP, NBP)])
    pltpu.sync_copy(h2, out_hbm.at[pl.ds(obase + 2 * NBP, NBP)])


def _sc_hist(l3, t3, topflat):
    mesh = plsc.VectorSubcoreMesh(core_axis_name="c", subcore_axis_name="s")
    k = functools.partial(
        pl.kernel,
        mesh=mesh,
        compiler_params=pltpu.CompilerParams(needs_layout_passes=False),
        out_type=jax.ShapeDtypeStruct((NW * 3 * NBP,), jnp.float32),
        scratch_types=[
            pltpu.VMEM((8, 512), jnp.float32),
            pltpu.VMEM((8, 512), jnp.float32),
            pltpu.VMEM((8, 512), jnp.float32),
            pltpu.VMEM((8, 512), jnp.float32),
            pltpu.VMEM((16,), jnp.int32),
            pltpu.VMEM((NBP,), jnp.float32),
            pltpu.VMEM((NBP,), jnp.float32),
            pltpu.VMEM((NBP,), jnp.float32),
            pltpu.SemaphoreType.DMA,
            pltpu.SemaphoreType.DMA,
        ],
    )(_sc_hist_body)
    return k(l3, t3, topflat)


def _fin_body(h_ref, p_ref, m_ref, o_ref):
    f32 = jnp.float32
    r = lax.broadcasted_iota(jnp.int32, (128, 128), 0)
    c = lax.broadcasted_iota(jnp.int32, (128, 128), 1)
    w_incl = (r >= c).astype(f32)
    w_strict = (r > c).astype(f32)

    total = jnp.zeros((), f32)
    for img in range(B):
        cc = img // 4
        s0 = (img % 4) * 4
        rows = [cc * 16 + s0 + j for j in range(4)]
        cnt = jnp.zeros((128, 128), f32)
        cntp = jnp.zeros((128, 128), f32)
        sm = jnp.zeros((128, 128), f32)
        for w in rows:
            cnt = cnt + h_ref[w, 0, :128, :]
            cntp = cntp + h_ref[w, 1, :128, :]
            sm = sm + h_ref[w, 2, :128, :]

        def suffix(x):
            s_in = jnp.dot(x, w_incl, preferred_element_type=f32)
            rowtot = s_in[:, :1]
            above = jnp.dot(w_strict.T, rowtot, preferred_element_type=f32)
            return s_in + above

        suf = suffix(cnt)
        sufp = suffix(cntp)
        a = suf - cnt
        ap = sufp - cntp
        p = p_ref[img, 0, 0]
        m = m_ref[img, 0, 0]
        tp = cntp
        tn = cnt - cntp
        sumpos = sm * cntp / jnp.maximum(cnt, 1.0)
        sumneg = sm - sumpos
        aa = p + a - ap
        contrib = (sumpos + sumneg * (p - ap - tp) / jnp.maximum(aa + tn, 1.0)
                   ) / jnp.maximum(aa, 1.0)
        loss = jnp.sum(contrib)
        loss = jnp.where(p == 0.0, jnp.maximum(m, 0.0), loss)
        total = total + loss
    o_ref[...] = jnp.broadcast_to(total * (1.0 / B), (1, 1))


def _finalize(h4, pc, mc):
    return pl.pallas_call(
        _fin_body,
        in_specs=[
            pl.BlockSpec((NW, 3, NBR, 128), lambda: (0, 0, 0, 0)),
            pl.BlockSpec((B, 1, 128), lambda: (0, 0, 0)),
            pl.BlockSpec((B, 1, 128), lambda: (0, 0, 0)),
        ],
        out_specs=pl.BlockSpec((1, 1), lambda: (0, 0)),
        out_shape=jax.ShapeDtypeStruct((1, 1), jnp.float32),
    )(h4, pc, mc)


def kernel(logits, targets):
    l4 = logits.reshape(B, 1, 512, 512)
    t4 = targets.reshape(B, 1, 512, 512)
    mc, pc = _prepass(l4, t4)
    top = lax.bitcast_convert_type(mc[:, 0, :16], jnp.int32).reshape(B * 16)
    hists = _sc_hist(logits.reshape(B, 512, 512), targets.reshape(B, 512, 512),
                     top)
    h4 = hists.reshape(NW, 3, NBR, 128)
    out = _finalize(h4, pc, mc)
    return out.reshape(())


# speedup vs baseline: 1.2914x; 1.0039x over previous
import functools

import jax
import jax.numpy as jnp
from jax import lax
from jax.experimental import pallas as pl
from jax.experimental.pallas import tpu as pltpu
from jax.experimental.pallas import tpu_sc as plsc

B = 8
N = 512 * 512
NB = 16384
NBR = 136
NBP = NBR * 128
SHIFT = 14
NW = 32
PER_W = N * B // NW
CH = 8192
NCHUNK = PER_W // CH


def _prep_body(l_ref, t_ref, m_ref, p_ref):
    l = l_ref[0, 0]
    t = t_ref[0, 0]
    tpos = jnp.where(t > 0.5, 1.0, 0.0).astype(jnp.float32)
    e = 1.0 - l * (2.0 * tpos - 1.0)
    m_ref[0, 0, :] = jnp.broadcast_to(jnp.max(e), (128,))
    p_ref[0, 0, :] = jnp.broadcast_to(jnp.sum(tpos), (128,))


def _prepass(logits4, targets4):
    return pl.pallas_call(
        _prep_body,
        grid=(B,),
        in_specs=[
            pl.BlockSpec((1, 1, 512, 512), lambda i: (i, 0, 0, 0)),
            pl.BlockSpec((1, 1, 512, 512), lambda i: (i, 0, 0, 0)),
        ],
        out_specs=[
            pl.BlockSpec((1, 1, 128), lambda i: (i, 0, 0)),
            pl.BlockSpec((1, 1, 128), lambda i: (i, 0, 0)),
        ],
        out_shape=[
            jax.ShapeDtypeStruct((B, 1, 128), jnp.float32),
            jax.ShapeDtypeStruct((B, 1, 128), jnp.float32),
        ],
    )(logits4, targets4)


def _sc_hist_body(l_hbm, t_hbm, top_hbm, out_hbm,
                  lb0, lb1, tb0, tb1, topv, h0, h1, h2, sem0, sem1):
    cid = lax.axis_index("c")
    sid = lax.axis_index("s")
    wid = cid * 16 + sid
    img = cid * 4 + sid // 4
    band0 = (sid % 4) * (NCHUNK * 16)

    zeros16 = jnp.zeros((16,), jnp.float32)

    @plsc.parallel_loop(0, NBP // 16, unroll=8)
    def _zero(j):
        h0[pl.ds(j * 16, 16)] = zeros16
        h1[pl.ds(j * 16, 16)] = zeros16
        h2[pl.ds(j * 16, 16)] = zeros16

    pltpu.sync_copy(top_hbm.at[pl.ds(img * 16, 16)], topv)
    top = topv[...]

    iota = lax.iota(jnp.int32, 16)
    nxt_idx = jnp.minimum(iota + 1, 15)
    not_last = iota < 15
    cnt_run = (iota + 1).astype(jnp.float32)
    neg_cnt_run = -cnt_run
    gdn = lax.GatherDimensionNumbers(
        offset_dims=(), collapsed_slice_dims=(0,), start_index_map=(0,))

    lbufs = (lb0, lb1)
    tbufs = (tb0, tb1)
    sems = (sem0, sem1)

    def _start(ci):
        rows = band0 + ci * 16
        lc = pltpu.async_copy(
            l_hbm.at[img, pl.ds(rows, 16), :], lbufs[ci % 2], sems[ci % 2])
        tc_ = pltpu.async_copy(
            t_hbm.at[img, pl.ds(rows, 16), :], tbufs[ci % 2], sems[ci % 2])
        return lc, tc_

    pend = _start(0)
    for ci in range(NCHUNK):
        lbuf = lbufs[ci % 2]
        tbuf = tbufs[ci % 2]
        nxt_pend = _start(ci + 1) if ci + 1 < NCHUNK else None
        pend[0].wait()
        pend[1].wait()
        pend = nxt_pend

        @plsc.parallel_loop(0, CH // 16, unroll=4)
        def _vec(v):
            r = v >> 5
            c = (v & 31) * 16
            l = lbuf[r, pl.ds(c, 16)]
            t = tbuf[r, pl.ds(c, 16)]
            lt = l * t
            e = (1.0 + l) - (lt + lt)
            es0 = jnp.maximum(e, 0.0)
            ebits = lax.bitcast_convert_type(es0, jnp.int32)
            d = lax.shift_right_arithmetic(top - ebits, SHIFT)
            bkt = jnp.minimum(jnp.maximum((NB - 1) - d, 0), NB - 1)
            bs, es = plsc.sort_key_val(bkt, es0)
            _, gs = plsc.sort_key_val(bkt, t)
            nxt = lax.gather(bs, nxt_idx[:, None], gdn, slice_sizes=(1,),
                             mode=lax.GatherScatterMode.PROMISE_IN_BOUNDS)
            brk = bs != nxt
            isend = brk | (~not_last)
            issub = brk & not_last
            ce = plsc.cumsum(es)
            cg = plsc.cumsum(gs)
            plsc.addupdate_scatter(h0, [bs], cnt_run, mask=isend)
            plsc.addupdate_scatter(h0, [nxt], neg_cnt_run, mask=issub)
            plsc.addupdate_scatter(h1, [bs], cg, mask=isend)
            plsc.addupdate_scatter(h1, [nxt], -cg, mask=issub)
            plsc.addupdate_scatter(h2, [bs], ce, mask=isend)
            plsc.addupdate_scatter(h2, [nxt], -ce, mask=issub)

    obase = wid * 3 * NBP
    pltpu.sync_copy(h0, out_hbm.at[pl.ds(obase, NBP)])
    pltpu.sync_copy(h1, out_hbm.at[pl.ds(obase + NBP, NBP)])
    pltpu.sync_copy(h2, out_hbm.at[pl.ds(obase + 2 * NBP, NBP)])


def _sc_hist(l3, t3, topflat):
    mesh = plsc.VectorSubcoreMesh(core_axis_name="c", subcore_axis_name="s")
    k = functools.partial(
        pl.kernel,
        mesh=mesh,
        compiler_params=pltpu.CompilerParams(needs_layout_passes=False),
        out_type=jax.ShapeDtypeStruct((NW * 3 * NBP,), jnp.float32),
        scratch_types=[
            pltpu.VMEM((16, 512), jnp.float32),
            pltpu.VMEM((16, 512), jnp.float32),
            pltpu.VMEM((16, 512), jnp.float32),
            pltpu.VMEM((16, 512), jnp.float32),
            pltpu.VMEM((16,), jnp.int32),
            pltpu.VMEM((NBP,), jnp.float32),
            pltpu.VMEM((NBP,), jnp.float32),
            pltpu.VMEM((NBP,), jnp.float32),
            pltpu.SemaphoreType.DMA,
            pltpu.SemaphoreType.DMA,
        ],
    )(_sc_hist_body)
    return k(l3, t3, topflat)


def _fin_body(h_ref, p_ref, m_ref, o_ref):
    f32 = jnp.float32
    r = lax.broadcasted_iota(jnp.int32, (128, 128), 0)
    c = lax.broadcasted_iota(jnp.int32, (128, 128), 1)
    w_incl = (r >= c).astype(f32)
    w_strict = (r > c).astype(f32)

    total = jnp.zeros((), f32)
    for img in range(B):
        cc = img // 4
        s0 = (img % 4) * 4
        rows = [cc * 16 + s0 + j for j in range(4)]
        cnt = jnp.zeros((128, 128), f32)
        cntp = jnp.zeros((128, 128), f32)
        sm = jnp.zeros((128, 128), f32)
        for w in rows:
            cnt = cnt + h_ref[w, 0, :128, :]
            cntp = cntp + h_ref[w, 1, :128, :]
            sm = sm + h_ref[w, 2, :128, :]

        def suffix(x):
            s_in = jnp.dot(x, w_incl, preferred_element_type=f32)
            rowtot = s_in[:, :1]
            above = jnp.dot(w_strict.T, rowtot, preferred_element_type=f32)
            return s_in + above

        suf = suffix(cnt)
        sufp = suffix(cntp)
        a = suf - cnt
        ap = sufp - cntp
        p = p_ref[img, 0, 0]
        m = m_ref[img, 0, 0]
        tp = cntp
        tn = cnt - cntp
        sumpos = sm * cntp / jnp.maximum(cnt, 1.0)
        sumneg = sm - sumpos
        aa = p + a - ap
        contrib = (sumpos + sumneg * (p - ap - tp) / jnp.maximum(aa + tn, 1.0)
                   ) / jnp.maximum(aa, 1.0)
        loss = jnp.sum(contrib)
        loss = jnp.where(p == 0.0, jnp.maximum(m, 0.0), loss)
        total = total + loss
    o_ref[...] = jnp.broadcast_to(total * (1.0 / B), (1, 1))


def _finalize(h4, pc, mc):
    return pl.pallas_call(
        _fin_body,
        in_specs=[
            pl.BlockSpec((NW, 3, NBR, 128), lambda: (0, 0, 0, 0)),
            pl.BlockSpec((B, 1, 128), lambda: (0, 0, 0)),
            pl.BlockSpec((B, 1, 128), lambda: (0, 0, 0)),
        ],
        out_specs=pl.BlockSpec((1, 1), lambda: (0, 0)),
        out_shape=jax.ShapeDtypeStruct((1, 1), jnp.float32),
    )(h4, pc, mc)


def kernel(logits, targets):
    l4 = logits.reshape(B, 1, 512, 512)
    t4 = targets.reshape(B, 1, 512, 512)
    mc, pc = _prepass(l4, t4)
    top = lax.bitcast_convert_type(mc[:, 0, :16], jnp.int32).reshape(B * 16)
    hists = _sc_hist(logits.reshape(B, 512, 512), targets.reshape(B, 512, 512),
                     top)
    h4 = hists.reshape(NW, 3, NBR, 128)
    out = _finalize(h4, pc, mc)
    return out.reshape(())


# speedup vs baseline: 1.3205x; 1.0225x over previous
import functools

import jax
import jax.numpy as jnp
from jax import lax
from jax.experimental import pallas as pl
from jax.experimental.pallas import tpu as pltpu
from jax.experimental.pallas import tpu_sc as plsc

B = 8
N = 512 * 512
NB = 16384
NBR = 136
NBP = NBR * 128
SHIFT = 14
NW = 32
PER_W = N * B // NW
CH = 8192
NCHUNK = PER_W // CH


def _prep_body(l_ref, t_ref, m_ref, p_ref):
    for i in range(2):
        l = l_ref[i, 0]
        t = t_ref[i, 0]
        tpos = jnp.where(t > 0.5, 1.0, 0.0).astype(jnp.float32)
        e = 1.0 - l * (2.0 * tpos - 1.0)
        m_ref[i, 0, :] = jnp.broadcast_to(jnp.max(e), (128,))
        p_ref[i, 0, :] = jnp.broadcast_to(jnp.sum(tpos), (128,))


def _prepass(logits4, targets4):
    return pl.pallas_call(
        _prep_body,
        grid=(B // 2,),
        in_specs=[
            pl.BlockSpec((2, 1, 512, 512), lambda i: (i, 0, 0, 0)),
            pl.BlockSpec((2, 1, 512, 512), lambda i: (i, 0, 0, 0)),
        ],
        out_specs=[
            pl.BlockSpec((2, 1, 128), lambda i: (i, 0, 0)),
            pl.BlockSpec((2, 1, 128), lambda i: (i, 0, 0)),
        ],
        out_shape=[
            jax.ShapeDtypeStruct((B, 1, 128), jnp.float32),
            jax.ShapeDtypeStruct((B, 1, 128), jnp.float32),
        ],
    )(logits4, targets4)


def _sc_hist_body(l_hbm, t_hbm, top_hbm, out_hbm,
                  lb0, lb1, tb0, tb1, topv, h0, h1, h2, sem0, sem1):
    cid = lax.axis_index("c")
    sid = lax.axis_index("s")
    wid = cid * 16 + sid
    img = cid * 4 + sid // 4
    band0 = (sid % 4) * (NCHUNK * 16)

    zeros16 = jnp.zeros((16,), jnp.float32)

    @plsc.parallel_loop(0, NBP // 16, unroll=8)
    def _zero(j):
        h0[pl.ds(j * 16, 16)] = zeros16
        h1[pl.ds(j * 16, 16)] = zeros16
        h2[pl.ds(j * 16, 16)] = zeros16

    pltpu.sync_copy(top_hbm.at[pl.ds(img * 16, 16)], topv)
    top = topv[...]

    iota = lax.iota(jnp.int32, 16)
    nxt_idx = jnp.minimum(iota + 1, 15)
    not_last = iota < 15
    cnt_run = (iota + 1).astype(jnp.float32)
    neg_cnt_run = -cnt_run
    gdn = lax.GatherDimensionNumbers(
        offset_dims=(), collapsed_slice_dims=(0,), start_index_map=(0,))

    lbufs = (lb0, lb1)
    tbufs = (tb0, tb1)
    sems = (sem0, sem1)

    def _start(ci):
        rows = band0 + ci * 16
        lc = pltpu.async_copy(
            l_hbm.at[img, pl.ds(rows, 16), :], lbufs[ci % 2], sems[ci % 2])
        tc_ = pltpu.async_copy(
            t_hbm.at[img, pl.ds(rows, 16), :], tbufs[ci % 2], sems[ci % 2])
        return lc, tc_

    pend = _start(0)
    for ci in range(NCHUNK):
        lbuf = lbufs[ci % 2]
        tbuf = tbufs[ci % 2]
        nxt_pend = _start(ci + 1) if ci + 1 < NCHUNK else None
        pend[0].wait()
        pend[1].wait()
        pend = nxt_pend

        @plsc.parallel_loop(0, CH // 16, unroll=4)
        def _vec(v):
            r = v >> 5
            c = (v & 31) * 16
            l = lbuf[r, pl.ds(c, 16)]
            t = tbuf[r, pl.ds(c, 16)]
            lt = l * t
            e = (1.0 + l) - (lt + lt)
            es0 = jnp.maximum(e, 0.0)
            ebits = lax.bitcast_convert_type(es0, jnp.int32)
            d = lax.shift_right_arithmetic(top - ebits, SHIFT)
            bkt = jnp.minimum(jnp.maximum((NB - 1) - d, 0), NB - 1)
            bs, es = plsc.sort_key_val(bkt, es0)
            _, gs = plsc.sort_key_val(bkt, t)
            nxt = lax.gather(bs, nxt_idx[:, None], gdn, slice_sizes=(1,),
                             mode=lax.GatherScatterMode.PROMISE_IN_BOUNDS)
            brk = bs != nxt
            isend = brk | (~not_last)
            issub = brk & not_last
            ce = plsc.cumsum(es)
            cg = plsc.cumsum(gs)
            plsc.addupdate_scatter(h0, [bs], cnt_run, mask=isend)
            plsc.addupdate_scatter(h0, [nxt], neg_cnt_run, mask=issub)
            plsc.addupdate_scatter(h1, [bs], cg, mask=isend)
            plsc.addupdate_scatter(h1, [nxt], -cg, mask=issub)
            plsc.addupdate_scatter(h2, [bs], ce, mask=isend)
            plsc.addupdate_scatter(h2, [nxt], -ce, mask=issub)

    obase = wid * 3 * NBP
    pltpu.sync_copy(h0, out_hbm.at[pl.ds(obase, NBP)])
    pltpu.sync_copy(h1, out_hbm.at[pl.ds(obase + NBP, NBP)])
    pltpu.sync_copy(h2, out_hbm.at[pl.ds(obase + 2 * NBP, NBP)])


def _sc_hist(l3, t3, topflat):
    mesh = plsc.VectorSubcoreMesh(core_axis_name="c", subcore_axis_name="s")
    k = functools.partial(
        pl.kernel,
        mesh=mesh,
        compiler_params=pltpu.CompilerParams(needs_layout_passes=False),
        out_type=jax.ShapeDtypeStruct((NW * 3 * NBP,), jnp.float32),
        scratch_types=[
            pltpu.VMEM((16, 512), jnp.float32),
            pltpu.VMEM((16, 512), jnp.float32),
            pltpu.VMEM((16, 512), jnp.float32),
            pltpu.VMEM((16, 512), jnp.float32),
            pltpu.VMEM((16,), jnp.int32),
            pltpu.VMEM((NBP,), jnp.float32),
            pltpu.VMEM((NBP,), jnp.float32),
            pltpu.VMEM((NBP,), jnp.float32),
            pltpu.SemaphoreType.DMA,
            pltpu.SemaphoreType.DMA,
        ],
    )(_sc_hist_body)
    return k(l3, t3, topflat)


def _fin_body(h_ref, p_ref, m_ref, o_ref):
    f32 = jnp.float32
    r = lax.broadcasted_iota(jnp.int32, (128, 128), 0)
    c = lax.broadcasted_iota(jnp.int32, (128, 128), 1)
    w_incl = (r >= c).astype(f32)
    w_strict = (r > c).astype(f32)

    total = jnp.zeros((), f32)
    for img in range(B):
        cc = img // 4
        s0 = (img % 4) * 4
        rows = [cc * 16 + s0 + j for j in range(4)]
        cnt = jnp.zeros((128, 128), f32)
        cntp = jnp.zeros((128, 128), f32)
        sm = jnp.zeros((128, 128), f32)
        for w in rows:
            cnt = cnt + h_ref[w, 0, :128, :]
            cntp = cntp + h_ref[w, 1, :128, :]
            sm = sm + h_ref[w, 2, :128, :]

        def suffix(x):
            s_in = jnp.dot(x, w_incl, preferred_element_type=f32)
            rowtot = s_in[:, :1]
            above = jnp.dot(w_strict.T, rowtot, preferred_element_type=f32)
            return s_in + above

        suf = suffix(cnt)
        sufp = suffix(cntp)
        a = suf - cnt
        ap = sufp - cntp
        p = p_ref[img, 0, 0]
        m = m_ref[img, 0, 0]
        tp = cntp
        tn = cnt - cntp
        sumpos = sm * cntp / jnp.maximum(cnt, 1.0)
        sumneg = sm - sumpos
        aa = p + a - ap
        contrib = (sumpos + sumneg * (p - ap - tp) / jnp.maximum(aa + tn, 1.0)
                   ) / jnp.maximum(aa, 1.0)
        loss = jnp.sum(contrib)
        loss = jnp.where(p == 0.0, jnp.maximum(m, 0.0), loss)
        total = total + loss
    o_ref[...] = jnp.broadcast_to(total * (1.0 / B), (1, 1))


def _finalize(h4, pc, mc):
    return pl.pallas_call(
        _fin_body,
        in_specs=[
            pl.BlockSpec((NW, 3, NBR, 128), lambda: (0, 0, 0, 0)),
            pl.BlockSpec((B, 1, 128), lambda: (0, 0, 0)),
            pl.BlockSpec((B, 1, 128), lambda: (0, 0, 0)),
        ],
        out_specs=pl.BlockSpec((1, 1), lambda: (0, 0)),
        out_shape=jax.ShapeDtypeStruct((1, 1), jnp.float32),
    )(h4, pc, mc)


def kernel(logits, targets):
    l4 = logits.reshape(B, 1, 512, 512)
    t4 = targets.reshape(B, 1, 512, 512)
    mc, pc = _prepass(l4, t4)
    top = lax.bitcast_convert_type(mc[:, 0, :16], jnp.int32).reshape(B * 16)
    hists = _sc_hist(logits.reshape(B, 512, 512), targets.reshape(B, 512, 512),
                     top)
    h4 = hists.reshape(NW, 3, NBR, 128)
    out = _finalize(h4, pc, mc)
    return out.reshape(())
